# baseline (speedup 1.0000x reference)
import functools

import jax
import jax.numpy as jnp
from jax import lax
from jax.experimental import pallas as pl
from jax.experimental.pallas import tpu as pltpu
from jax.experimental.pallas import tpu_sc as plsc

_N = 10000
_E = 320000
_C = 128
_H = 8
_D = 16
_HD = _H * _D
_LAT = 32

_NCORES = 2
_NSUB = 16
_BLK = 64
_NBLOCKS = _E // _BLK
_CORE_B = (2500, 2500)
_ROWS_PER_TILE = 632
_N_ACC = _ROWS_PER_TILE * _NSUB
_ROW_CHUNKS = [(k * _BLK, _BLK) for k in range(9)] + [(576, 56)]
_AW = 144

_ROWS1 = 1000
_ROWS2 = 1000


def _head_masks(shape_rows, shape_cols):
    row = lax.broadcasted_iota(jnp.int32, (shape_rows, shape_cols), 0)
    col = lax.broadcasted_iota(jnp.int32, (shape_rows, shape_cols), 1)
    return col == row // _D, col == _H + row // _D


def _prep_body(x_ref, wg_ref, as_ref, ad_ref, ha_ref, adt_ref):
    h = jnp.dot(x_ref[...], wg_ref[...], preferred_element_type=jnp.float32)
    m_src, m_dst = _head_masks(_C, 2 * _H)
    a_s = jnp.broadcast_to(as_ref[...], (_C, 2 * _H))
    a_d = jnp.broadcast_to(ad_ref[...], (_C, 2 * _H))
    zero = jnp.zeros((), jnp.float32)
    ma = jnp.where(m_src, a_s, zero) + jnp.where(m_dst, a_d, zero)
    mb = jnp.where(m_src, a_d, zero) + jnp.where(m_dst, a_s, zero)
    aa = jnp.dot(h, ma, preferred_element_type=jnp.float32)
    ab = jnp.dot(h, mb, preferred_element_type=jnp.float32)
    ha_ref[...] = jnp.concatenate([h, aa], axis=1)
    adt_ref[...] = ab


def _run_prep(x, W_gat, a_src_col, a_dst_col):
    grid = _N // _ROWS1
    return pl.pallas_call(
        _prep_body,
        grid=(grid,),
        in_specs=[
            pl.BlockSpec((_ROWS1, _C), lambda i: (i, 0)),
            pl.BlockSpec((_C, _HD), lambda i: (0, 0)),
            pl.BlockSpec((_C, 1), lambda i: (0, 0)),
            pl.BlockSpec((_C, 1), lambda i: (0, 0)),
        ],
        out_specs=[
            pl.BlockSpec((_ROWS1, _AW), lambda i: (i, 0)),
            pl.BlockSpec((_ROWS1, 2 * _H), lambda i: (i, 0)),
        ],
        out_shape=[
            jax.ShapeDtypeStruct((_N, _AW), jnp.float32),
            jax.ShapeDtypeStruct((_N, 2 * _H), jnp.float32),
        ],
    )(x, W_gat, a_src_col, a_dst_col)


_sc_mesh = plsc.VectorSubcoreMesh(core_axis_name="c", subcore_axis_name="s")

_GATHER_DNUMS = lax.GatherDimensionNumbers(
    offset_dims=(), collapsed_slice_dims=(0,), start_index_map=(0,))


@functools.partial(
    pl.kernel,
    out_type=jax.ShapeDtypeStruct((_NCORES, _N_ACC, _AW), jnp.float32),
    mesh=_sc_mesh,
    scratch_types=[
        [pltpu.VMEM((_BLK, _AW), jnp.float32) for _ in range(3)],
        [pltpu.VMEM((_BLK, 2 * _H), jnp.float32) for _ in range(3)],
        [pltpu.VMEM((_BLK,), jnp.int32) for _ in range(6)],
        [pltpu.VMEM((_BLK,), jnp.int32) for _ in range(6)],
        pltpu.VMEM_SHARED((_N_ACC, _AW), jnp.float32),
        [pltpu.SemaphoreType.DMA for _ in range(3)],
        [pltpu.SemaphoreType.DMA for _ in range(3)],
        [pltpu.SemaphoreType.DMA for _ in range(6)],
    ],
    compiler_params=pltpu.CompilerParams(use_tc_tiling_on_sc=False),
)
def _sc_gat(ha_hbm, ad_hbm, ei_hbm, out_hbm,
            ha_bufs, ad_bufs, si_bufs, di_bufs, acc_sh, sgs, sss, sis):
    c = lax.axis_index("c")
    s = lax.axis_index("s")

    bc = jnp.where(c == 0, _CORE_B[0], _CORE_B[1])
    core_base = jnp.where(c == 0, 0, _CORE_B[0])
    q, rem = bc // _NSUB, bc % _NSUB
    nblk = q + jnp.where(s < rem, 1, 0)
    tile_base = core_base + s * q + jnp.minimum(s, rem)
    e0 = tile_base * _BLK
    nb6 = (nblk // 6) * 6

    def issue_idx(blk, p6):
        g = e0 + blk * _BLK
        pltpu.async_copy(ei_hbm.at[0, g // 128, pl.ds(g % 128, _BLK)],
                         si_bufs[p6], sis[p6])
        pltpu.async_copy(ei_hbm.at[1, g // 128, pl.ds(g % 128, _BLK)],
                         di_bufs[p6], sis[p6])

    def wait_idx(p6):
        pltpu.make_async_copy(ei_hbm.at[0, 0, pl.ds(0, _BLK)],
                              si_bufs[p6], sis[p6]).wait()
        pltpu.make_async_copy(ei_hbm.at[1, 0, pl.ds(0, _BLK)],
                              di_bufs[p6], sis[p6]).wait()

    def issue_gather(p3, p6):
        pltpu.async_copy(ha_hbm.at[si_bufs[p6]], ha_bufs[p3], sgs[p3])
        pltpu.async_copy(ad_hbm.at[di_bufs[p6]], ad_bufs[p3], sgs[p3])

    def wait_gather(p3):
        pltpu.make_async_copy(ha_hbm.at[si_bufs[0]], ha_bufs[p3], sgs[p3]).wait()
        pltpu.make_async_copy(ad_hbm.at[di_bufs[0]], ad_bufs[p3], sgs[p3]).wait()

    def wait_scatter(p3):
        pltpu.make_async_copy(ha_bufs[p3], acc_sh.at[di_bufs[0]], sss[p3]).wait()

    def compute_block(ha_b, ad_b):
        @plsc.parallel_loop(0, _BLK, unroll=2)
        def _(e):
            av = ha_b[e, pl.ds(_HD, 16)] + ad_b[e, :]
            av = jnp.maximum(av, av * 0.2)
            w = jnp.exp(av)
            for i in range(_H):
                bi = lax.gather(
                    w, jnp.full((16, 1), i, jnp.int32), _GATHER_DNUMS,
                    (1,), mode=lax.GatherScatterMode.PROMISE_IN_BOUNDS)
                ha_b[e, pl.ds(i * 16, 16)] = ha_b[e, pl.ds(i * 16, 16)] * bi
            ha_b[e, pl.ds(_HD, 16)] = w

    @pl.loop(0, _BLK)
    def _(e):
        for j in range(_AW // 16):
            ha_bufs[0][e, pl.ds(j * 16, 16)] = jnp.zeros((16,), jnp.float32)

    row0 = s * _ROWS_PER_TILE

    for off, sz in _ROW_CHUNKS:
        pltpu.sync_copy(ha_bufs[0].at[pl.ds(0, sz)],
                        acc_sh.at[pl.ds(row0 + off, sz)])

    plsc.subcore_barrier()

    for b in range(3):
        issue_idx(b, b)
    for b in range(2):
        wait_idx(b)
        issue_gather(b, b)

    @pl.loop(0, nb6, step=6)
    def _(blk0):
        for j in range(6):
            blk = blk0 + j
            p3 = j % 3
            ha_b, ad_b = ha_bufs[p3], ad_bufs[p3]
            wait_gather(p3)
            compute_block(ha_b, ad_b)
            pltpu.async_copy(ha_b, acc_sh.at[di_bufs[j]], sss[p3], add=True)

            @pl.when(blk + 2 < nb6)
            def _():
                @pl.when(blk >= 1)
                def _():
                    wait_scatter((j + 2) % 3)
                wait_idx((j + 2) % 6)
                issue_gather((j + 2) % 3, (j + 2) % 6)

            @pl.when(blk + 3 < nb6)
            def _():
                issue_idx(blk + 3, (j + 3) % 6)

    for p3 in range(3):
        wait_scatter(p3)

    @pl.loop(nb6, nblk)
    def _(blk):
        g = e0 + blk * _BLK
        pltpu.sync_copy(ei_hbm.at[0, g // 128, pl.ds(g % 128, _BLK)], si_bufs[0])
        pltpu.sync_copy(ei_hbm.at[1, g // 128, pl.ds(g % 128, _BLK)], di_bufs[0])
        pltpu.async_copy(ha_hbm.at[si_bufs[0]], ha_bufs[0], sgs[0]).wait()
        pltpu.async_copy(ad_hbm.at[di_bufs[0]], ad_bufs[0], sgs[0]).wait()
        compute_block(ha_bufs[0], ad_bufs[0])
        pltpu.sync_copy(ha_bufs[0], acc_sh.at[di_bufs[0]], add=True)

    plsc.subcore_barrier()

    for off, sz in _ROW_CHUNKS:
        r0 = row0 + off
        pltpu.sync_copy(acc_sh.at[pl.ds(r0, sz)], ha_bufs[0].at[pl.ds(0, sz)])
        pltpu.sync_copy(ha_bufs[0].at[pl.ds(0, sz)], out_hbm.at[c, pl.ds(r0, sz)])


def _post_body(acc_ref, ha_ref, bg_ref, we_ref, be_ref, wd_ref, bd_ref,
               wo_ref, bo_ref, o_ref):
    a = acc_ref[0] + acc_ref[1]
    hh = ha_ref[:, :_HD]
    av = ha_ref[:, _HD:_HD + _H] + ha_ref[:, _HD + _H:_AW]
    av = jnp.maximum(av, av * 0.2)
    w_self = jnp.exp(av)

    rrow = lax.broadcasted_iota(jnp.int32, (_H, _HD), 0)
    rcol = lax.broadcasted_iota(jnp.int32, (_H, _HD), 1)
    rmat = jnp.where(rrow == rcol // _D, 1.0, 0.0).astype(jnp.float32)

    num = a[:, :_HD] + jnp.dot(w_self, rmat,
                               preferred_element_type=jnp.float32) * hh
    den = a[:, _HD:_HD + _H] + w_self
    recip = 1.0 / (den + 1e-16)
    denw = jnp.dot(recip, rmat, preferred_element_type=jnp.float32)
    g = num * denw + bg_ref[...]
    g = jnp.where(g > 0, g, jnp.exp(jnp.minimum(g, 0.0)) - 1.0)
    e1 = jnp.dot(g, we_ref[...], preferred_element_type=jnp.float32) + be_ref[...]
    d1 = jnp.dot(e1, wd_ref[...], preferred_element_type=jnp.float32) + bd_ref[...]
    h2 = jnp.where(d1 > 0, d1, jnp.exp(jnp.minimum(d1, 0.0)) - 1.0)
    o_ref[...] = jnp.dot(h2, wo_ref[...], preferred_element_type=jnp.float32) + bo_ref[...]


def _run_post(acc, ha, b_gat, W_enc, b_enc, W_dec, b_dec, W_out, b_out):
    grid = _N // _ROWS2
    return pl.pallas_call(
        _post_body,
        grid=(grid,),
        in_specs=[
            pl.BlockSpec((_NCORES, _ROWS2, _AW), lambda i: (0, i, 0)),
            pl.BlockSpec((_ROWS2, _AW), lambda i: (i, 0)),
            pl.BlockSpec((1, _HD), lambda i: (0, 0)),
            pl.BlockSpec((_HD, _LAT), lambda i: (0, 0)),
            pl.BlockSpec((1, _LAT), lambda i: (0, 0)),
            pl.BlockSpec((_LAT, _HD), lambda i: (0, 0)),
            pl.BlockSpec((1, _HD), lambda i: (0, 0)),
            pl.BlockSpec((_HD, _C), lambda i: (0, 0)),
            pl.BlockSpec((1, _C), lambda i: (0, 0)),
        ],
        out_specs=pl.BlockSpec((_ROWS2, _C), lambda i: (i, 0)),
        out_shape=jax.ShapeDtypeStruct((_N, _C), jnp.float32),
    )(acc, ha, b_gat, W_enc, b_enc, W_dec, b_dec, W_out, b_out)


def kernel(x, edge_index, W_gat, att_src, att_dst, b_gat,
           W_enc, b_enc, W_dec, b_dec, W_out, b_out):
    ha, ad = _run_prep(x, W_gat,
                       att_src.reshape(_HD, 1), att_dst.reshape(_HD, 1))
    acc = _sc_gat(ha, ad, edge_index.reshape(2, _E // 128, 128))
    return _run_post(acc, ha,
                     b_gat.reshape(1, _HD), W_enc, b_enc.reshape(1, _LAT),
                     W_dec, b_dec.reshape(1, _HD), W_out, b_out.reshape(1, _C))


# speedup vs baseline: 1.0105x; 1.0105x over previous
import functools

import jax
import jax.numpy as jnp
from jax import lax
from jax.experimental import pallas as pl
from jax.experimental.pallas import tpu as pltpu
from jax.experimental.pallas import tpu_sc as plsc

_N = 10000
_E = 320000
_C = 128
_H = 8
_D = 16
_HD = _H * _D
_LAT = 32

_NCORES = 2
_NSUB = 16
_BLK = 64
_NBLOCKS = _E // _BLK
_CORE_B = (2500, 2500)
_ROWS_PER_TILE = 632
_N_ACC = _ROWS_PER_TILE * _NSUB
_ROW_CHUNKS = [(k * _BLK, _BLK) for k in range(9)] + [(576, 56)]
_AW = 144

_ROWS1 = 1000
_ROWS2 = 1000


def _head_masks(shape_rows, shape_cols):
    row = lax.broadcasted_iota(jnp.int32, (shape_rows, shape_cols), 0)
    col = lax.broadcasted_iota(jnp.int32, (shape_rows, shape_cols), 1)
    return col == row // _D, col == _H + row // _D


def _prep_body(x_ref, wg_ref, wp_ref, as_ref, ad_ref,
               ha_ref, hbp_ref, sa_ref, adt_ref):
    h = jnp.dot(x_ref[...], wg_ref[...], preferred_element_type=jnp.float32)
    hp = jnp.dot(x_ref[...], wp_ref[...], preferred_element_type=jnp.float32)
    m_src, m_dst = _head_masks(_C, 2 * _H)
    a_s = jnp.broadcast_to(as_ref[...], (_C, 2 * _H))
    a_d = jnp.broadcast_to(ad_ref[...], (_C, 2 * _H))
    zero = jnp.zeros((), jnp.float32)
    ma = jnp.where(m_src, a_s, zero) + jnp.where(m_dst, a_d, zero)
    mb = jnp.where(m_src, a_d, zero) + jnp.where(m_dst, a_s, zero)
    aa = jnp.dot(h, ma, preferred_element_type=jnp.float32)
    ab = jnp.dot(h, mb, preferred_element_type=jnp.float32)
    ha_ref[...] = jnp.concatenate([h, aa], axis=1)
    hbp_ref[...] = hp.astype(jnp.bfloat16)
    sa_ref[...] = aa
    adt_ref[...] = ab


def _run_prep(x, W_gat, W_perm, a_src_col, a_dst_col):
    grid = _N // _ROWS1
    return pl.pallas_call(
        _prep_body,
        grid=(grid,),
        in_specs=[
            pl.BlockSpec((_ROWS1, _C), lambda i: (i, 0)),
            pl.BlockSpec((_C, _HD), lambda i: (0, 0)),
            pl.BlockSpec((_C, _HD), lambda i: (0, 0)),
            pl.BlockSpec((_C, 1), lambda i: (0, 0)),
            pl.BlockSpec((_C, 1), lambda i: (0, 0)),
        ],
        out_specs=[
            pl.BlockSpec((_ROWS1, _AW), lambda i: (i, 0)),
            pl.BlockSpec((_ROWS1, _HD), lambda i: (i, 0)),
            pl.BlockSpec((_ROWS1, 2 * _H), lambda i: (i, 0)),
            pl.BlockSpec((_ROWS1, 2 * _H), lambda i: (i, 0)),
        ],
        out_shape=[
            jax.ShapeDtypeStruct((_N, _AW), jnp.float32),
            jax.ShapeDtypeStruct((_N, _HD), jnp.bfloat16),
            jax.ShapeDtypeStruct((_N, 2 * _H), jnp.float32),
            jax.ShapeDtypeStruct((_N, 2 * _H), jnp.float32),
        ],
    )(x, W_gat, W_perm, a_src_col, a_dst_col)


_sc_mesh = plsc.VectorSubcoreMesh(core_axis_name="c", subcore_axis_name="s")

_GATHER_DNUMS = lax.GatherDimensionNumbers(
    offset_dims=(), collapsed_slice_dims=(0,), start_index_map=(0,))


@functools.partial(
    pl.kernel,
    out_type=jax.ShapeDtypeStruct((_NCORES, _N_ACC, _AW), jnp.float32),
    mesh=_sc_mesh,
    scratch_types=[
        [pltpu.VMEM((_BLK, _HD), jnp.bfloat16) for _ in range(3)],
        [pltpu.VMEM((_BLK, 2 * _H), jnp.float32) for _ in range(3)],
        [pltpu.VMEM((_BLK, 2 * _H), jnp.float32) for _ in range(3)],
        [pltpu.VMEM((_BLK, _AW), jnp.float32) for _ in range(2)],
        [pltpu.VMEM((_BLK,), jnp.int32) for _ in range(6)],
        [pltpu.VMEM((_BLK,), jnp.int32) for _ in range(6)],
        pltpu.VMEM_SHARED((_N_ACC, _AW), jnp.float32),
        [pltpu.SemaphoreType.DMA for _ in range(3)],
        [pltpu.SemaphoreType.DMA for _ in range(2)],
        [pltpu.SemaphoreType.DMA for _ in range(6)],
    ],
    compiler_params=pltpu.CompilerParams(use_tc_tiling_on_sc=False,
                                         needs_layout_passes=False),
)
def _sc_gat(hb_hbm, sa_hbm, ad_hbm, ei_hbm, out_hbm,
            hb_bufs, sa_bufs, ad_bufs, msg_bufs, si_bufs, di_bufs,
            acc_sh, sgs, sss, sis):
    c = lax.axis_index("c")
    s = lax.axis_index("s")

    bc = jnp.where(c == 0, _CORE_B[0], _CORE_B[1])
    core_base = jnp.where(c == 0, 0, _CORE_B[0])
    q, rem = bc // _NSUB, bc % _NSUB
    nblk = q + jnp.where(s < rem, 1, 0)
    tile_base = core_base + s * q + jnp.minimum(s, rem)
    e0 = tile_base * _BLK
    nb6 = (nblk // 6) * 6

    def issue_idx(blk, p6):
        g = e0 + blk * _BLK
        pltpu.async_copy(ei_hbm.at[0, g // 128, pl.ds(g % 128, _BLK)],
                         si_bufs[p6], sis[p6])
        pltpu.async_copy(ei_hbm.at[1, g // 128, pl.ds(g % 128, _BLK)],
                         di_bufs[p6], sis[p6])

    def wait_idx(p6):
        pltpu.make_async_copy(ei_hbm.at[0, 0, pl.ds(0, _BLK)],
                              si_bufs[p6], sis[p6]).wait()
        pltpu.make_async_copy(ei_hbm.at[1, 0, pl.ds(0, _BLK)],
                              di_bufs[p6], sis[p6]).wait()

    def issue_gather(p3, p6):
        pltpu.async_copy(hb_hbm.at[si_bufs[p6]], hb_bufs[p3], sgs[p3])
        pltpu.async_copy(sa_hbm.at[si_bufs[p6]], sa_bufs[p3], sgs[p3])
        pltpu.async_copy(ad_hbm.at[di_bufs[p6]], ad_bufs[p3], sgs[p3])

    def wait_gather(p3):
        pltpu.make_async_copy(hb_hbm.at[si_bufs[0]], hb_bufs[p3], sgs[p3]).wait()
        pltpu.make_async_copy(sa_hbm.at[si_bufs[0]], sa_bufs[p3], sgs[p3]).wait()
        pltpu.make_async_copy(ad_hbm.at[di_bufs[0]], ad_bufs[p3], sgs[p3]).wait()

    def wait_scatter(m2):
        pltpu.make_async_copy(msg_bufs[m2], acc_sh.at[di_bufs[0]],
                              sss[m2]).wait()

    def compute_block(hb_b, sa_b, ad_b, msg_b):
        @plsc.parallel_loop(0, _BLK, unroll=2)
        def _(e):
            av = sa_b[e, :] + ad_b[e, :]
            av = jnp.maximum(av, av * 0.2)
            w = jnp.exp(av)
            for i in range(_H // 2):
                v = hb_b[e, pl.ds(32 * i, 32)]
                h0, h1 = plsc.unpack(v, format=plsc.PackFormat.INTERLEAVED)
                b0 = lax.gather(
                    w, jnp.full((16, 1), 2 * i, jnp.int32), _GATHER_DNUMS,
                    (1,), mode=lax.GatherScatterMode.PROMISE_IN_BOUNDS)
                b1 = lax.gather(
                    w, jnp.full((16, 1), 2 * i + 1, jnp.int32), _GATHER_DNUMS,
                    (1,), mode=lax.GatherScatterMode.PROMISE_IN_BOUNDS)
                msg_b[e, pl.ds(32 * i, 16)] = h0 * b0
                msg_b[e, pl.ds(32 * i + 16, 16)] = h1 * b1
            msg_b[e, pl.ds(_HD, 16)] = w

    @pl.loop(0, _BLK)
    def _(e):
        for j in range(_AW // 16):
            msg_bufs[0][e, pl.ds(j * 16, 16)] = jnp.zeros((16,), jnp.float32)

    row0 = s * _ROWS_PER_TILE

    for off, sz in _ROW_CHUNKS:
        pltpu.sync_copy(msg_bufs[0].at[pl.ds(0, sz)],
                        acc_sh.at[pl.ds(row0 + off, sz)])

    plsc.subcore_barrier()

    for b in range(3):
        issue_idx(b, b)
    for b in range(2):
        wait_idx(b)
        issue_gather(b, b)

    @pl.loop(0, nb6, step=6)
    def _(blk0):
        for j in range(6):
            blk = blk0 + j
            p3 = j % 3
            m2 = j % 2
            wait_gather(p3)

            @pl.when(blk >= 2)
            def _():
                wait_scatter(m2)

            compute_block(hb_bufs[p3], sa_bufs[p3], ad_bufs[p3], msg_bufs[m2])
            pltpu.async_copy(msg_bufs[m2], acc_sh.at[di_bufs[j]], sss[m2],
                             add=True)

            @pl.when(blk + 2 < nb6)
            def _():
                wait_idx((j + 2) % 6)
                issue_gather((j + 2) % 3, (j + 2) % 6)

            @pl.when(blk + 3 < nb6)
            def _():
                issue_idx(blk + 3, (j + 3) % 6)

    for m2 in range(2):
        wait_scatter(m2)

    @pl.loop(nb6, nblk)
    def _(blk):
        g = e0 + blk * _BLK
        pltpu.sync_copy(ei_hbm.at[0, g // 128, pl.ds(g % 128, _BLK)], si_bufs[0])
        pltpu.sync_copy(ei_hbm.at[1, g // 128, pl.ds(g % 128, _BLK)], di_bufs[0])
        pltpu.async_copy(hb_hbm.at[si_bufs[0]], hb_bufs[0], sgs[0]).wait()
        pltpu.async_copy(sa_hbm.at[si_bufs[0]], sa_bufs[0], sgs[0]).wait()
        pltpu.async_copy(ad_hbm.at[di_bufs[0]], ad_bufs[0], sgs[0]).wait()
        compute_block(hb_bufs[0], sa_bufs[0], ad_bufs[0], msg_bufs[0])
        pltpu.sync_copy(msg_bufs[0], acc_sh.at[di_bufs[0]], add=True)

    plsc.subcore_barrier()

    for off, sz in _ROW_CHUNKS:
        r0 = row0 + off
        pltpu.sync_copy(acc_sh.at[pl.ds(r0, sz)], msg_bufs[0].at[pl.ds(0, sz)])
        pltpu.sync_copy(msg_bufs[0].at[pl.ds(0, sz)],
                        out_hbm.at[c, pl.ds(r0, sz)])


def _post_body(acc_ref, ha_ref, bg_ref, we_ref, be_ref, wd_ref, bd_ref,
               wo_ref, bo_ref, o_ref):
    a = acc_ref[0] + acc_ref[1]
    hh = ha_ref[:, :_HD]
    av = ha_ref[:, _HD:_HD + _H] + ha_ref[:, _HD + _H:_AW]
    av = jnp.maximum(av, av * 0.2)
    w_self = jnp.exp(av)

    rrow = lax.broadcasted_iota(jnp.int32, (_H, _HD), 0)
    rcol = lax.broadcasted_iota(jnp.int32, (_H, _HD), 1)
    rmat = jnp.where(rrow == rcol // _D, 1.0, 0.0).astype(jnp.float32)

    num = a[:, :_HD] + jnp.dot(w_self, rmat,
                               preferred_element_type=jnp.float32) * hh
    den = a[:, _HD:_HD + _H] + w_self
    recip = 1.0 / (den + 1e-16)
    denw = jnp.dot(recip, rmat, preferred_element_type=jnp.float32)
    g = num * denw + bg_ref[...]
    g = jnp.where(g > 0, g, jnp.exp(jnp.minimum(g, 0.0)) - 1.0)
    e1 = jnp.dot(g, we_ref[...], preferred_element_type=jnp.float32) + be_ref[...]
    d1 = jnp.dot(e1, wd_ref[...], preferred_element_type=jnp.float32) + bd_ref[...]
    h2 = jnp.where(d1 > 0, d1, jnp.exp(jnp.minimum(d1, 0.0)) - 1.0)
    o_ref[...] = jnp.dot(h2, wo_ref[...], preferred_element_type=jnp.float32) + bo_ref[...]


def _run_post(acc, ha, b_gat, W_enc, b_enc, W_dec, b_dec, W_out, b_out):
    grid = _N // _ROWS2
    return pl.pallas_call(
        _post_body,
        grid=(grid,),
        in_specs=[
            pl.BlockSpec((_NCORES, _ROWS2, _AW), lambda i: (0, i, 0)),
            pl.BlockSpec((_ROWS2, _AW), lambda i: (i, 0)),
            pl.BlockSpec((1, _HD), lambda i: (0, 0)),
            pl.BlockSpec((_HD, _LAT), lambda i: (0, 0)),
            pl.BlockSpec((1, _LAT), lambda i: (0, 0)),
            pl.BlockSpec((_LAT, _HD), lambda i: (0, 0)),
            pl.BlockSpec((1, _HD), lambda i: (0, 0)),
            pl.BlockSpec((_HD, _C), lambda i: (0, 0)),
            pl.BlockSpec((1, _C), lambda i: (0, 0)),
        ],
        out_specs=pl.BlockSpec((_ROWS2, _C), lambda i: (i, 0)),
        out_shape=jax.ShapeDtypeStruct((_N, _C), jnp.float32),
    )(acc, ha, b_gat, W_enc, b_enc, W_dec, b_dec, W_out, b_out)


def kernel(x, edge_index, W_gat, att_src, att_dst, b_gat,
           W_enc, b_enc, W_dec, b_dec, W_out, b_out):
    perm = [0] * _HD
    for g in range(_H // 2):
        for j in range(_D):
            perm[32 * g + 2 * j] = 32 * g + j
            perm[32 * g + 2 * j + 1] = 32 * g + _D + j
    W_perm = W_gat[:, jnp.array(perm, dtype=jnp.int32)]

    ha, hbp, sa, ad = _run_prep(x, W_gat, W_perm,
                                att_src.reshape(_HD, 1),
                                att_dst.reshape(_HD, 1))
    acc = _sc_gat(hbp, sa, ad, edge_index.reshape(2, _E // 128, 128))
    return _run_post(acc, ha,
                     b_gat.reshape(1, _HD), W_enc, b_enc.reshape(1, _LAT),
                     W_dec, b_dec.reshape(1, _HD), W_out, b_out.reshape(1, _C))


# speedup vs baseline: 1.0162x; 1.0056x over previous
import functools

import jax
import jax.numpy as jnp
from jax import lax
from jax.experimental import pallas as pl
from jax.experimental.pallas import tpu as pltpu
from jax.experimental.pallas import tpu_sc as plsc

_N = 10000
_E = 320000
_C = 128
_H = 8
_D = 16
_HD = _H * _D
_LAT = 32

_NCORES = 2
_NSUB = 16
_BLK = 64
_NBLOCKS = _E // _BLK
_CORE_B = (2500, 2500)
_ROWS_PER_TILE = 632
_N_ACC = _ROWS_PER_TILE * _NSUB
_ROW_CHUNKS = [(k * _BLK, _BLK) for k in range(9)] + [(576, 56)]
_AW = 144

_ROWS1 = 1000
_ROWS2 = 1000


def _head_masks(shape_rows, shape_cols):
    row = lax.broadcasted_iota(jnp.int32, (shape_rows, shape_cols), 0)
    col = lax.broadcasted_iota(jnp.int32, (shape_rows, shape_cols), 1)
    return col == row // _D, col == _H + row // _D


def _prep_body(x_ref, wg_ref, wp_ref, as_ref, ad_ref,
               ha_ref, hbp_ref, sa_ref, adt_ref):
    h = jnp.dot(x_ref[...], wg_ref[...], preferred_element_type=jnp.float32)
    hp = jnp.dot(x_ref[...], wp_ref[...], preferred_element_type=jnp.float32)
    m_src, m_dst = _head_masks(_C, 2 * _H)
    a_s = jnp.broadcast_to(as_ref[...], (_C, 2 * _H))
    a_d = jnp.broadcast_to(ad_ref[...], (_C, 2 * _H))
    zero = jnp.zeros((), jnp.float32)
    ma = jnp.where(m_src, a_s, zero) + jnp.where(m_dst, a_d, zero)
    mb = jnp.where(m_src, a_d, zero) + jnp.where(m_dst, a_s, zero)
    aa = jnp.dot(h, ma, preferred_element_type=jnp.float32)
    ab = jnp.dot(h, mb, preferred_element_type=jnp.float32)
    ha_ref[...] = jnp.concatenate([h, aa], axis=1)
    hbp_ref[...] = hp.astype(jnp.bfloat16)
    sa_ref[...] = aa
    adt_ref[...] = ab


def _run_prep(x, W_gat, W_perm, a_src_col, a_dst_col):
    grid = _N // _ROWS1
    return pl.pallas_call(
        _prep_body,
        grid=(grid,),
        in_specs=[
            pl.BlockSpec((_ROWS1, _C), lambda i: (i, 0)),
            pl.BlockSpec((_C, _HD), lambda i: (0, 0)),
            pl.BlockSpec((_C, _HD), lambda i: (0, 0)),
            pl.BlockSpec((_C, 1), lambda i: (0, 0)),
            pl.BlockSpec((_C, 1), lambda i: (0, 0)),
        ],
        out_specs=[
            pl.BlockSpec((_ROWS1, _AW), lambda i: (i, 0)),
            pl.BlockSpec((_ROWS1, _HD), lambda i: (i, 0)),
            pl.BlockSpec((_ROWS1, 2 * _H), lambda i: (i, 0)),
            pl.BlockSpec((_ROWS1, 2 * _H), lambda i: (i, 0)),
        ],
        out_shape=[
            jax.ShapeDtypeStruct((_N, _AW), jnp.float32),
            jax.ShapeDtypeStruct((_N, _HD), jnp.bfloat16),
            jax.ShapeDtypeStruct((_N, 2 * _H), jnp.float32),
            jax.ShapeDtypeStruct((_N, 2 * _H), jnp.float32),
        ],
    )(x, W_gat, W_perm, a_src_col, a_dst_col)


_sc_mesh = plsc.VectorSubcoreMesh(core_axis_name="c", subcore_axis_name="s")

_GATHER_DNUMS = lax.GatherDimensionNumbers(
    offset_dims=(), collapsed_slice_dims=(0,), start_index_map=(0,))


@functools.partial(
    pl.kernel,
    out_type=jax.ShapeDtypeStruct((_NCORES, _N_ACC, _AW), jnp.float32),
    mesh=_sc_mesh,
    scratch_types=[
        [pltpu.VMEM((_BLK, _HD), jnp.bfloat16) for _ in range(3)],
        [pltpu.VMEM((_BLK, 2 * _H), jnp.float32) for _ in range(3)],
        [pltpu.VMEM((_BLK, 2 * _H), jnp.float32) for _ in range(3)],
        [pltpu.VMEM((_BLK, _AW), jnp.float32) for _ in range(2)],
        [pltpu.VMEM((_BLK,), jnp.int32) for _ in range(6)],
        [pltpu.VMEM((_BLK,), jnp.int32) for _ in range(6)],
        pltpu.VMEM_SHARED((_N_ACC, _AW), jnp.float32),
        [pltpu.SemaphoreType.DMA for _ in range(3)],
        [pltpu.SemaphoreType.DMA for _ in range(2)],
        [pltpu.SemaphoreType.DMA for _ in range(6)],
    ],
    compiler_params=pltpu.CompilerParams(use_tc_tiling_on_sc=False,
                                         needs_layout_passes=False),
)
def _sc_gat(hb_hbm, sa_hbm, ad_hbm, ei_hbm, out_hbm,
            hb_bufs, sa_bufs, ad_bufs, msg_bufs, si_bufs, di_bufs,
            acc_sh, sgs, sss, sis):
    c = lax.axis_index("c")
    s = lax.axis_index("s")

    bc = jnp.where(c == 0, _CORE_B[0], _CORE_B[1])
    core_base = jnp.where(c == 0, 0, _CORE_B[0])
    q, rem = bc // _NSUB, bc % _NSUB
    nblk = q + jnp.where(s < rem, 1, 0)
    tile_base = core_base + s * q + jnp.minimum(s, rem)
    e0 = tile_base * _BLK
    nb6 = (nblk // 6) * 6

    def issue_idx(blk, p6):
        g = e0 + blk * _BLK
        pltpu.async_copy(ei_hbm.at[0, g // 128, pl.ds(g % 128, _BLK)],
                         si_bufs[p6], sis[p6])
        pltpu.async_copy(ei_hbm.at[1, g // 128, pl.ds(g % 128, _BLK)],
                         di_bufs[p6], sis[p6])

    def wait_idx(p6):
        pltpu.make_async_copy(ei_hbm.at[0, 0, pl.ds(0, _BLK)],
                              si_bufs[p6], sis[p6]).wait()
        pltpu.make_async_copy(ei_hbm.at[1, 0, pl.ds(0, _BLK)],
                              di_bufs[p6], sis[p6]).wait()

    def issue_gather(p3, p6):
        pltpu.async_copy(hb_hbm.at[si_bufs[p6]], hb_bufs[p3], sgs[p3])
        pltpu.async_copy(sa_hbm.at[si_bufs[p6]], sa_bufs[p3], sgs[p3])
        pltpu.async_copy(ad_hbm.at[di_bufs[p6]], ad_bufs[p3], sgs[p3])

    def wait_gather(p3):
        pltpu.make_async_copy(hb_hbm.at[si_bufs[0]], hb_bufs[p3], sgs[p3]).wait()
        pltpu.make_async_copy(sa_hbm.at[si_bufs[0]], sa_bufs[p3], sgs[p3]).wait()
        pltpu.make_async_copy(ad_hbm.at[di_bufs[0]], ad_bufs[p3], sgs[p3]).wait()

    def wait_scatter(m2):
        pltpu.make_async_copy(msg_bufs[m2], acc_sh.at[di_bufs[0]],
                              sss[m2]).wait()

    def compute_block(hb_b, sa_b, ad_b, msg_b):
        @plsc.parallel_loop(0, _BLK, unroll=2)
        def _(e):
            av = sa_b[e, :] + ad_b[e, :]
            av = jnp.maximum(av, av * 0.2)
            w = jnp.exp(av)
            msg_b[e, pl.ds(_HD, 16)] = w
            for i in range(_H // 2):
                v = hb_b[e, pl.ds(32 * i, 32)]
                h0, h1 = plsc.unpack(v, format=plsc.PackFormat.INTERLEAVED)
                msg_b[e, pl.ds(32 * i, 16)] = h0 * w[2 * i]
                msg_b[e, pl.ds(32 * i + 16, 16)] = h1 * w[2 * i + 1]

    @pl.loop(0, _BLK)
    def _(e):
        for j in range(_AW // 16):
            msg_bufs[0][e, pl.ds(j * 16, 16)] = jnp.zeros((16,), jnp.float32)

    row0 = s * _ROWS_PER_TILE

    for off, sz in _ROW_CHUNKS:
        pltpu.sync_copy(msg_bufs[0].at[pl.ds(0, sz)],
                        acc_sh.at[pl.ds(row0 + off, sz)])

    plsc.subcore_barrier()

    for b in range(3):
        issue_idx(b, b)
    for b in range(2):
        wait_idx(b)
        issue_gather(b, b)

    @pl.loop(0, nb6, step=6)
    def _(blk0):
        for j in range(6):
            blk = blk0 + j
            p3 = j % 3
            m2 = j % 2
            wait_gather(p3)

            @pl.when(blk >= 2)
            def _():
                wait_scatter(m2)

            compute_block(hb_bufs[p3], sa_bufs[p3], ad_bufs[p3], msg_bufs[m2])
            pltpu.async_copy(msg_bufs[m2], acc_sh.at[di_bufs[j]], sss[m2],
                             add=True)

            @pl.when(blk + 2 < nb6)
            def _():
                wait_idx((j + 2) % 6)
                issue_gather((j + 2) % 3, (j + 2) % 6)

            @pl.when(blk + 3 < nb6)
            def _():
                issue_idx(blk + 3, (j + 3) % 6)

    for m2 in range(2):
        wait_scatter(m2)

    @pl.loop(nb6, nblk)
    def _(blk):
        g = e0 + blk * _BLK
        pltpu.sync_copy(ei_hbm.at[0, g // 128, pl.ds(g % 128, _BLK)], si_bufs[0])
        pltpu.sync_copy(ei_hbm.at[1, g // 128, pl.ds(g % 128, _BLK)], di_bufs[0])
        pltpu.async_copy(hb_hbm.at[si_bufs[0]], hb_bufs[0], sgs[0]).wait()
        pltpu.async_copy(sa_hbm.at[si_bufs[0]], sa_bufs[0], sgs[0]).wait()
        pltpu.async_copy(ad_hbm.at[di_bufs[0]], ad_bufs[0], sgs[0]).wait()
        compute_block(hb_bufs[0], sa_bufs[0], ad_bufs[0], msg_bufs[0])
        pltpu.sync_copy(msg_bufs[0], acc_sh.at[di_bufs[0]], add=True)

    plsc.subcore_barrier()

    for off, sz in _ROW_CHUNKS:
        r0 = row0 + off
        pltpu.sync_copy(acc_sh.at[pl.ds(r0, sz)], msg_bufs[0].at[pl.ds(0, sz)])
        pltpu.sync_copy(msg_bufs[0].at[pl.ds(0, sz)],
                        out_hbm.at[c, pl.ds(r0, sz)])


def _post_body(acc_ref, ha_ref, bg_ref, we_ref, be_ref, wd_ref, bd_ref,
               wo_ref, bo_ref, o_ref):
    a = acc_ref[0] + acc_ref[1]
    hh = ha_ref[:, :_HD]
    av = ha_ref[:, _HD:_HD + _H] + ha_ref[:, _HD + _H:_AW]
    av = jnp.maximum(av, av * 0.2)
    w_self = jnp.exp(av)

    rrow = lax.broadcasted_iota(jnp.int32, (_H, _HD), 0)
    rcol = lax.broadcasted_iota(jnp.int32, (_H, _HD), 1)
    rmat = jnp.where(rrow == rcol // _D, 1.0, 0.0).astype(jnp.float32)

    num = a[:, :_HD] + jnp.dot(w_self, rmat,
                               preferred_element_type=jnp.float32) * hh
    den = a[:, _HD:_HD + _H] + w_self
    recip = 1.0 / (den + 1e-16)
    denw = jnp.dot(recip, rmat, preferred_element_type=jnp.float32)
    g = num * denw + bg_ref[...]
    g = jnp.where(g > 0, g, jnp.exp(jnp.minimum(g, 0.0)) - 1.0)
    e1 = jnp.dot(g, we_ref[...], preferred_element_type=jnp.float32) + be_ref[...]
    d1 = jnp.dot(e1, wd_ref[...], preferred_element_type=jnp.float32) + bd_ref[...]
    h2 = jnp.where(d1 > 0, d1, jnp.exp(jnp.minimum(d1, 0.0)) - 1.0)
    o_ref[...] = jnp.dot(h2, wo_ref[...], preferred_element_type=jnp.float32) + bo_ref[...]


def _run_post(acc, ha, b_gat, W_enc, b_enc, W_dec, b_dec, W_out, b_out):
    grid = _N // _ROWS2
    return pl.pallas_call(
        _post_body,
        grid=(grid,),
        in_specs=[
            pl.BlockSpec((_NCORES, _ROWS2, _AW), lambda i: (0, i, 0)),
            pl.BlockSpec((_ROWS2, _AW), lambda i: (i, 0)),
            pl.BlockSpec((1, _HD), lambda i: (0, 0)),
            pl.BlockSpec((_HD, _LAT), lambda i: (0, 0)),
            pl.BlockSpec((1, _LAT), lambda i: (0, 0)),
            pl.BlockSpec((_LAT, _HD), lambda i: (0, 0)),
            pl.BlockSpec((1, _HD), lambda i: (0, 0)),
            pl.BlockSpec((_HD, _C), lambda i: (0, 0)),
            pl.BlockSpec((1, _C), lambda i: (0, 0)),
        ],
        out_specs=pl.BlockSpec((_ROWS2, _C), lambda i: (i, 0)),
        out_shape=jax.ShapeDtypeStruct((_N, _C), jnp.float32),
    )(acc, ha, b_gat, W_enc, b_enc, W_dec, b_dec, W_out, b_out)


def kernel(x, edge_index, W_gat, att_src, att_dst, b_gat,
           W_enc, b_enc, W_dec, b_dec, W_out, b_out):
    perm = [0] * _HD
    for g in range(_H // 2):
        for j in range(_D):
            perm[32 * g + 2 * j] = 32 * g + j
            perm[32 * g + 2 * j + 1] = 32 * g + _D + j
    W_perm = W_gat[:, jnp.array(perm, dtype=jnp.int32)]

    ha, hbp, sa, ad = _run_prep(x, W_gat, W_perm,
                                att_src.reshape(_HD, 1),
                                att_dst.reshape(_HD, 1))
    acc = _sc_gat(hbp, sa, ad, edge_index.reshape(2, _E // 128, 128))
    return _run_post(acc, ha,
                     b_gat.reshape(1, _HD), W_enc, b_enc.reshape(1, _LAT),
                     W_dec, b_dec.reshape(1, _HD), W_out, b_out.reshape(1, _C))


# speedup vs baseline: 1.0484x; 1.0318x over previous
import functools

import jax
import jax.numpy as jnp
from jax import lax
from jax.experimental import pallas as pl
from jax.experimental.pallas import tpu as pltpu
from jax.experimental.pallas import tpu_sc as plsc

_N = 10000
_E = 320000
_C = 128
_H = 8
_D = 16
_HD = _H * _D
_LAT = 32

_NCORES = 2
_NSUB = 16
_BLK = 64
_NBLOCKS = _E // _BLK
_CORE_B = (2500, 2500)
_ROWS_PER_TILE = 632
_N_ACC = _ROWS_PER_TILE * _NSUB
_ROW_CHUNKS = [(k * _BLK, _BLK) for k in range(9)] + [(576, 56)]
_AW = 144

_ROWS1 = 2000
_ROWS2 = 2000


def _head_masks(shape_rows, shape_cols):
    row = lax.broadcasted_iota(jnp.int32, (shape_rows, shape_cols), 0)
    col = lax.broadcasted_iota(jnp.int32, (shape_rows, shape_cols), 1)
    return col == row // _D, col == _H + row // _D


def _prep_body(x_ref, wg_ref, wp_ref, as_ref, ad_ref,
               ha_ref, hbp_ref, sa_ref, adt_ref):
    h = jnp.dot(x_ref[...], wg_ref[...], preferred_element_type=jnp.float32)
    hp = jnp.dot(x_ref[...], wp_ref[...], preferred_element_type=jnp.float32)
    m_src, m_dst = _head_masks(_C, 2 * _H)
    a_s = jnp.broadcast_to(as_ref[...], (_C, 2 * _H))
    a_d = jnp.broadcast_to(ad_ref[...], (_C, 2 * _H))
    zero = jnp.zeros((), jnp.float32)
    ma = jnp.where(m_src, a_s, zero) + jnp.where(m_dst, a_d, zero)
    mb = jnp.where(m_src, a_d, zero) + jnp.where(m_dst, a_s, zero)
    aa = jnp.dot(h, ma, preferred_element_type=jnp.float32)
    ab = jnp.dot(h, mb, preferred_element_type=jnp.float32)
    ha_ref[...] = jnp.concatenate([h, aa], axis=1)
    hbp_ref[...] = hp.astype(jnp.bfloat16)
    sa_ref[...] = aa
    adt_ref[...] = ab


def _run_prep(x, W_gat, W_perm, a_src_col, a_dst_col):
    grid = _N // _ROWS1
    return pl.pallas_call(
        _prep_body,
        grid=(grid,),
        in_specs=[
            pl.BlockSpec((_ROWS1, _C), lambda i: (i, 0)),
            pl.BlockSpec((_C, _HD), lambda i: (0, 0)),
            pl.BlockSpec((_C, _HD), lambda i: (0, 0)),
            pl.BlockSpec((_C, 1), lambda i: (0, 0)),
            pl.BlockSpec((_C, 1), lambda i: (0, 0)),
        ],
        out_specs=[
            pl.BlockSpec((_ROWS1, _AW), lambda i: (i, 0)),
            pl.BlockSpec((_ROWS1, _HD), lambda i: (i, 0)),
            pl.BlockSpec((_ROWS1, 2 * _H), lambda i: (i, 0)),
            pl.BlockSpec((_ROWS1, 2 * _H), lambda i: (i, 0)),
        ],
        out_shape=[
            jax.ShapeDtypeStruct((_N, _AW), jnp.float32),
            jax.ShapeDtypeStruct((_N, _HD), jnp.bfloat16),
            jax.ShapeDtypeStruct((_N, 2 * _H), jnp.float32),
            jax.ShapeDtypeStruct((_N, 2 * _H), jnp.float32),
        ],
    )(x, W_gat, W_perm, a_src_col, a_dst_col)


_sc_mesh = plsc.VectorSubcoreMesh(core_axis_name="c", subcore_axis_name="s")

_GATHER_DNUMS = lax.GatherDimensionNumbers(
    offset_dims=(), collapsed_slice_dims=(0,), start_index_map=(0,))


@functools.partial(
    pl.kernel,
    out_type=jax.ShapeDtypeStruct((_NCORES, _N_ACC, _AW), jnp.float32),
    mesh=_sc_mesh,
    scratch_types=[
        [pltpu.VMEM((_BLK, _HD), jnp.bfloat16) for _ in range(3)],
        [pltpu.VMEM((_BLK, 2 * _H), jnp.float32) for _ in range(3)],
        [pltpu.VMEM((_BLK, 2 * _H), jnp.float32) for _ in range(3)],
        [pltpu.VMEM((_BLK, _AW), jnp.float32) for _ in range(2)],
        [pltpu.VMEM((_BLK,), jnp.int32) for _ in range(6)],
        [pltpu.VMEM((_BLK,), jnp.int32) for _ in range(6)],
        pltpu.VMEM_SHARED((_N_ACC, _AW), jnp.float32),
        [pltpu.SemaphoreType.DMA for _ in range(3)],
        [pltpu.SemaphoreType.DMA for _ in range(2)],
        [pltpu.SemaphoreType.DMA for _ in range(6)],
    ],
    compiler_params=pltpu.CompilerParams(use_tc_tiling_on_sc=False,
                                         needs_layout_passes=False),
)
def _sc_gat(hb_hbm, sa_hbm, ad_hbm, ei_hbm, out_hbm,
            hb_bufs, sa_bufs, ad_bufs, msg_bufs, si_bufs, di_bufs,
            acc_sh, sgs, sss, sis):
    c = lax.axis_index("c")
    s = lax.axis_index("s")

    bc = jnp.where(c == 0, _CORE_B[0], _CORE_B[1])
    core_base = jnp.where(c == 0, 0, _CORE_B[0])
    q, rem = bc // _NSUB, bc % _NSUB
    nblk = q + jnp.where(s < rem, 1, 0)
    tile_base = core_base + s * q + jnp.minimum(s, rem)
    e0 = tile_base * _BLK
    nb6 = (nblk // 6) * 6

    def issue_idx(blk, p6):
        g = e0 + blk * _BLK
        pltpu.async_copy(ei_hbm.at[0, g // 128, pl.ds(g % 128, _BLK)],
                         si_bufs[p6], sis[p6])
        pltpu.async_copy(ei_hbm.at[1, g // 128, pl.ds(g % 128, _BLK)],
                         di_bufs[p6], sis[p6])

    def wait_idx(p6):
        pltpu.make_async_copy(ei_hbm.at[0, 0, pl.ds(0, _BLK)],
                              si_bufs[p6], sis[p6]).wait()
        pltpu.make_async_copy(ei_hbm.at[1, 0, pl.ds(0, _BLK)],
                              di_bufs[p6], sis[p6]).wait()

    def issue_gather(p3, p6):
        pltpu.async_copy(hb_hbm.at[si_bufs[p6]], hb_bufs[p3], sgs[p3])
        pltpu.async_copy(sa_hbm.at[si_bufs[p6]], sa_bufs[p3], sgs[p3])
        pltpu.async_copy(ad_hbm.at[di_bufs[p6]], ad_bufs[p3], sgs[p3])

    def wait_gather(p3):
        pltpu.make_async_copy(hb_hbm.at[si_bufs[0]], hb_bufs[p3], sgs[p3]).wait()
        pltpu.make_async_copy(sa_hbm.at[si_bufs[0]], sa_bufs[p3], sgs[p3]).wait()
        pltpu.make_async_copy(ad_hbm.at[di_bufs[0]], ad_bufs[p3], sgs[p3]).wait()

    def wait_scatter(m2):
        pltpu.make_async_copy(msg_bufs[m2], acc_sh.at[di_bufs[0]],
                              sss[m2]).wait()

    def compute_block(hb_b, sa_b, ad_b, msg_b):
        @plsc.parallel_loop(0, _BLK, unroll=2)
        def _(e):
            av = sa_b[e, :] + ad_b[e, :]
            av = jnp.maximum(av, av * 0.2)
            w = jnp.exp(av)
            msg_b[e, pl.ds(_HD, 16)] = w
            for i in range(_H // 2):
                v = hb_b[e, pl.ds(32 * i, 32)]
                h0, h1 = plsc.unpack(v, format=plsc.PackFormat.INTERLEAVED)
                msg_b[e, pl.ds(32 * i, 16)] = h0 * w[2 * i]
                msg_b[e, pl.ds(32 * i + 16, 16)] = h1 * w[2 * i + 1]

    @pl.loop(0, _BLK)
    def _(e):
        for j in range(_AW // 16):
            msg_bufs[0][e, pl.ds(j * 16, 16)] = jnp.zeros((16,), jnp.float32)

    row0 = s * _ROWS_PER_TILE

    for off, sz in _ROW_CHUNKS:
        pltpu.sync_copy(msg_bufs[0].at[pl.ds(0, sz)],
                        acc_sh.at[pl.ds(row0 + off, sz)])

    plsc.subcore_barrier()

    for b in range(3):
        issue_idx(b, b)
    for b in range(2):
        wait_idx(b)
        issue_gather(b, b)

    @pl.loop(0, nb6, step=6)
    def _(blk0):
        for j in range(6):
            blk = blk0 + j
            p3 = j % 3
            m2 = j % 2
            wait_gather(p3)

            @pl.when(blk >= 2)
            def _():
                wait_scatter(m2)

            compute_block(hb_bufs[p3], sa_bufs[p3], ad_bufs[p3], msg_bufs[m2])
            pltpu.async_copy(msg_bufs[m2], acc_sh.at[di_bufs[j]], sss[m2],
                             add=True)

            @pl.when(blk + 2 < nb6)
            def _():
                wait_idx((j + 2) % 6)
                issue_gather((j + 2) % 3, (j + 2) % 6)

            @pl.when(blk + 3 < nb6)
            def _():
                issue_idx(blk + 3, (j + 3) % 6)

    for m2 in range(2):
        wait_scatter(m2)

    @pl.loop(nb6, nblk)
    def _(blk):
        g = e0 + blk * _BLK
        pltpu.sync_copy(ei_hbm.at[0, g // 128, pl.ds(g % 128, _BLK)], si_bufs[0])
        pltpu.sync_copy(ei_hbm.at[1, g // 128, pl.ds(g % 128, _BLK)], di_bufs[0])
        pltpu.async_copy(hb_hbm.at[si_bufs[0]], hb_bufs[0], sgs[0]).wait()
        pltpu.async_copy(sa_hbm.at[si_bufs[0]], sa_bufs[0], sgs[0]).wait()
        pltpu.async_copy(ad_hbm.at[di_bufs[0]], ad_bufs[0], sgs[0]).wait()
        compute_block(hb_bufs[0], sa_bufs[0], ad_bufs[0], msg_bufs[0])
        pltpu.sync_copy(msg_bufs[0], acc_sh.at[di_bufs[0]], add=True)

    plsc.subcore_barrier()

    for off, sz in _ROW_CHUNKS:
        r0 = row0 + off
        pltpu.sync_copy(acc_sh.at[pl.ds(r0, sz)], msg_bufs[0].at[pl.ds(0, sz)])
        pltpu.sync_copy(msg_bufs[0].at[pl.ds(0, sz)],
                        out_hbm.at[c, pl.ds(r0, sz)])


def _post_body(acc_ref, ha_ref, bg_ref, we_ref, be_ref, wd_ref, bd_ref,
               wo_ref, bo_ref, o_ref):
    a = acc_ref[0] + acc_ref[1]
    hh = ha_ref[:, :_HD]
    av = ha_ref[:, _HD:_HD + _H] + ha_ref[:, _HD + _H:_AW]
    av = jnp.maximum(av, av * 0.2)
    w_self = jnp.exp(av)

    rrow = lax.broadcasted_iota(jnp.int32, (_H, _HD), 0)
    rcol = lax.broadcasted_iota(jnp.int32, (_H, _HD), 1)
    rmat = jnp.where(rrow == rcol // _D, 1.0, 0.0).astype(jnp.float32)

    num = a[:, :_HD] + jnp.dot(w_self, rmat,
                               preferred_element_type=jnp.float32) * hh
    den = a[:, _HD:_HD + _H] + w_self
    recip = 1.0 / (den + 1e-16)
    denw = jnp.dot(recip, rmat, preferred_element_type=jnp.float32)
    g = num * denw + bg_ref[...]
    g = jnp.where(g > 0, g, jnp.exp(jnp.minimum(g, 0.0)) - 1.0)
    e1 = jnp.dot(g, we_ref[...], preferred_element_type=jnp.float32) + be_ref[...]
    d1 = jnp.dot(e1, wd_ref[...], preferred_element_type=jnp.float32) + bd_ref[...]
    h2 = jnp.where(d1 > 0, d1, jnp.exp(jnp.minimum(d1, 0.0)) - 1.0)
    o_ref[...] = jnp.dot(h2, wo_ref[...], preferred_element_type=jnp.float32) + bo_ref[...]


def _run_post(acc, ha, b_gat, W_enc, b_enc, W_dec, b_dec, W_out, b_out):
    grid = _N // _ROWS2
    return pl.pallas_call(
        _post_body,
        grid=(grid,),
        in_specs=[
            pl.BlockSpec((_NCORES, _ROWS2, _AW), lambda i: (0, i, 0)),
            pl.BlockSpec((_ROWS2, _AW), lambda i: (i, 0)),
            pl.BlockSpec((1, _HD), lambda i: (0, 0)),
            pl.BlockSpec((_HD, _LAT), lambda i: (0, 0)),
            pl.BlockSpec((1, _LAT), lambda i: (0, 0)),
            pl.BlockSpec((_LAT, _HD), lambda i: (0, 0)),
            pl.BlockSpec((1, _HD), lambda i: (0, 0)),
            pl.BlockSpec((_HD, _C), lambda i: (0, 0)),
            pl.BlockSpec((1, _C), lambda i: (0, 0)),
        ],
        out_specs=pl.BlockSpec((_ROWS2, _C), lambda i: (i, 0)),
        out_shape=jax.ShapeDtypeStruct((_N, _C), jnp.float32),
    )(acc, ha, b_gat, W_enc, b_enc, W_dec, b_dec, W_out, b_out)


def kernel(x, edge_index, W_gat, att_src, att_dst, b_gat,
           W_enc, b_enc, W_dec, b_dec, W_out, b_out):
    perm = [0] * _HD
    for g in range(_H // 2):
        for j in range(_D):
            perm[32 * g + 2 * j] = 32 * g + j
            perm[32 * g + 2 * j + 1] = 32 * g + _D + j
    W_perm = W_gat[:, jnp.array(perm, dtype=jnp.int32)]

    ha, hbp, sa, ad = _run_prep(x, W_gat, W_perm,
                                att_src.reshape(_HD, 1),
                                att_dst.reshape(_HD, 1))
    acc = _sc_gat(hbp, sa, ad, edge_index.reshape(2, _E // 128, 128))
    return _run_post(acc, ha,
                     b_gat.reshape(1, _HD), W_enc, b_enc.reshape(1, _LAT),
                     W_dec, b_dec.reshape(1, _HD), W_out, b_out.reshape(1, _C))
